# Initial kernel scaffold; baseline (speedup 1.0000x reference)
import jax
import jax.numpy as jnp
from jax.experimental import pallas as pl


def kernel(x, edge_index, rel_type, weight, bias):
    raise NotImplementedError("write your pallas kernel here")


# speedup vs baseline: 6.5284x; 6.5284x over previous
import functools

import jax
import jax.numpy as jnp
from jax import lax
from jax.experimental import pallas as pl
from jax.experimental.pallas import tpu as pltpu
from jax.experimental.pallas import tpu_sc as plsc

NC = 2
NS = 16
NW = NC * NS
LANES = 16

EDGE_CHUNK = 128


def _xw_body(x_ref, w_ref, y_ref):
    y_ref[0] = jnp.dot(x_ref[...], w_ref[0], preferred_element_type=jnp.float32)


def _relation_transform(x, weight):
    n, in_feat = x.shape
    num_rels, _, out_feat = weight.shape
    blk = 400
    return pl.pallas_call(
        _xw_body,
        grid=(num_rels, n // blk),
        in_specs=[
            pl.BlockSpec((blk, in_feat), lambda r, i: (i, 0)),
            pl.BlockSpec((1, in_feat, out_feat), lambda r, i: (r, 0, 0)),
        ],
        out_specs=pl.BlockSpec((1, blk, out_feat), lambda r, i: (r, i, 0)),
        out_shape=jax.ShapeDtypeStruct((num_rels, n, out_feat), jnp.float32),
    )(x, weight)


def _make_scatter(n, e, out_feat):
    nchunk = e // EDGE_CHUNK
    n_pad = ((n + 8 * NS - 1) // (8 * NS)) * (8 * NS)
    rows_per_tile = n_pad // NS
    zrows = 128
    mesh = plsc.VectorSubcoreMesh(
        core_axis_name="c", subcore_axis_name="s", num_cores=NC, num_subcores=NS
    )

    @functools.partial(
        pl.kernel,
        mesh=mesh,
        out_type=jax.ShapeDtypeStruct((NC, n_pad, out_feat), jnp.float32),
        scratch_types=[
            pltpu.VMEM((EDGE_CHUNK,), jnp.int32),
            pltpu.VMEM((EDGE_CHUNK,), jnp.int32),
            pltpu.VMEM((EDGE_CHUNK,), jnp.int32),
            pltpu.VMEM((EDGE_CHUNK,), jnp.int32),
            pltpu.VMEM((EDGE_CHUNK, out_feat), jnp.float32),
            pltpu.VMEM((zrows, out_feat), jnp.float32),
            pltpu.VMEM_SHARED((n_pad, out_feat), jnp.float32),
            pltpu.SemaphoreType.DMA,
        ],
    )
    def scatter_kernel(y_hbm, src_hbm, rel_hbm, dst_hbm, out_hbm,
                       srcb, relb, dstb, idxb, rows, zbuf, acc, sem):
        c = lax.axis_index("c")
        s = lax.axis_index("s")
        wid = s * NC + c

        def zero_row(i, _):
            def zero_col(j, _):
                zbuf[i, pl.ds(j * LANES, LANES)] = jnp.zeros((LANES,), jnp.float32)
                return 0
            return lax.fori_loop(0, out_feat // LANES, zero_col, 0)

        lax.fori_loop(0, zrows, zero_row, 0)
        r0 = s * rows_per_tile
        for t in range(rows_per_tile // zrows):
            pltpu.sync_copy(zbuf, acc.at[pl.ds(r0 + t * zrows, zrows)])
        plsc.subcore_barrier()

        nk = nchunk // NW + jnp.where(wid < nchunk % NW, 1, 0)

        def body(k, _):
            off = pl.multiple_of((k * NW + wid) * EDGE_CHUNK, EDGE_CHUNK)
            pltpu.sync_copy(src_hbm.at[pl.ds(off, EDGE_CHUNK)], srcb)
            pltpu.sync_copy(rel_hbm.at[pl.ds(off, EDGE_CHUNK)], relb)
            pltpu.sync_copy(dst_hbm.at[pl.ds(off, EDGE_CHUNK)], dstb)
            for j in range(EDGE_CHUNK // LANES):
                sl = pl.ds(j * LANES, LANES)
                idxb[sl] = relb[sl] * n + srcb[sl]
            pltpu.async_copy(y_hbm.at[idxb], rows, sem).wait()
            pltpu.sync_copy(rows, acc.at[dstb], add=True)
            return 0

        lax.fori_loop(0, nk, body, 0)
        plsc.subcore_barrier()

        pltpu.sync_copy(acc.at[pl.ds(r0, rows_per_tile)],
                        out_hbm.at[c, pl.ds(r0, rows_per_tile)])

    return scatter_kernel


def _fin_body(p_ref, b_ref, o_ref):
    o_ref[...] = jnp.maximum(p_ref[0] + p_ref[1] + b_ref[...], 0.0)


def _finalize(partials, bias, n):
    out_feat = partials.shape[-1]
    blk = 2000
    return pl.pallas_call(
        _fin_body,
        grid=(n // blk,),
        in_specs=[
            pl.BlockSpec((NC, blk, out_feat), lambda i: (0, i, 0)),
            pl.BlockSpec((1, out_feat), lambda i: (0, 0)),
        ],
        out_specs=pl.BlockSpec((blk, out_feat), lambda i: (i, 0)),
        out_shape=jax.ShapeDtypeStruct((n, out_feat), jnp.float32),
    )(partials, bias.reshape(1, out_feat))


def kernel(x, edge_index, rel_type, weight, bias):
    n, _ = x.shape
    e = rel_type.shape[0]
    num_rels, _, out_feat = weight.shape

    y = _relation_transform(x, weight)
    y_flat = y.reshape(num_rels * n, out_feat)

    src = edge_index[0]
    dst = edge_index[1]
    partials = _make_scatter(n, e, out_feat)(y_flat, src, rel_type, dst)
    return _finalize(partials, bias, n)
